# baseline (speedup 1.0000x reference)
import functools

import jax
import jax.numpy as jnp
from jax import lax
from jax.experimental import pallas as pl
from jax.experimental.pallas import tpu as pltpu
from jax.experimental.pallas import tpu_sc as plsc

_N = 10000
_E = 320000
_D = 128
_L = 16


def _sc_indegree_mask(edge_index):
    mesh = plsc.VectorSubcoreMesh(core_axis_name="c", subcore_axis_name="s")
    nc, ns = mesh.num_cores, mesh.num_subcores
    nw = nc * ns
    n_pad = ((_N + ns * _L - 1) // (ns * _L)) * (ns * _L)
    seg = n_pad // ns

    n_tiles = _E // 128
    tpw = -(-n_tiles // nw)
    lo_scale = n_tiles - tpw

    @functools.partial(
        pl.kernel,
        mesh=mesh,
        out_type=jax.ShapeDtypeStruct((nc * n_pad,), jnp.float32),
        compiler_params=pltpu.CompilerParams(needs_layout_passes=False),
        scratch_types=[
            pltpu.VMEM((2, tpw * 128), jnp.int32),
            pltpu.VMEM((n_pad,), jnp.float32),
            pltpu.VMEM((ns, seg), jnp.float32),
            pltpu.VMEM_SHARED((ns, ns, seg), jnp.float32),
            pltpu.SemaphoreType.DMA,
            pltpu.SemaphoreType.DMA,
            pltpu.SemaphoreType.DMA,
        ],
    )
    def k(ei_hbm, out_hbm, idx_v, mask_v, merge_v, stage_s,
          sem_a, sem_b, st_sem):
        c = lax.axis_index("c")
        s = lax.axis_index("s")
        wid = c * ns + s
        lo = wid * lo_scale // (nw - 1)
        zero16 = jnp.zeros((_L,), jnp.float32)
        one16 = jnp.ones((_L,), jnp.float32)

        ta = tpw // 2
        tb = tpw - ta
        cp_a = pltpu.async_copy(
            ei_hbm.at[:, pl.ds(lo * 128, ta * 128)],
            idx_v.at[:, pl.ds(0, ta * 128)], sem_a)
        cp_b = pltpu.async_copy(
            ei_hbm.at[:, pl.ds((lo + ta) * 128, tb * 128)],
            idx_v.at[:, pl.ds(ta * 128, tb * 128)], sem_b)

        def init_body(i, carry):
            for u in range(8):
                mask_v[pl.ds((i * 8 + u) * _L, _L)] = zero16
            return carry

        lax.fori_loop(0, n_pad // _L // 8, init_body, 0)

        def scat_body(i, carry):
            for u in range(8):
                idx = idx_v[1, pl.ds((i * 8 + u) * _L, _L)]
                plsc.store_scatter(mask_v, [idx], one16)
            return carry

        cp_a.wait()
        lax.fori_loop(0, ta * 128 // _L // 8, scat_body, 0)
        cp_b.wait()
        lax.fori_loop(ta * 128 // _L // 8, tpw * 128 // _L // 8,
                      scat_body, 0)

        copies = [
            pltpu.async_copy(mask_v.at[pl.ds(chunk * seg, seg)],
                             stage_s.at[chunk, s], st_sem)
            for chunk in range(ns)
        ]
        for cp in copies:
            cp.wait()
        plsc.subcore_barrier()
        pltpu.sync_copy(stage_s.at[s], merge_v)

        def merge_body(j, carry):
            acc = zero16
            for t in range(ns):
                acc = acc + merge_v[t, pl.ds(j * _L, _L)]
            mask_v[pl.ds(j * _L, _L)] = acc
            return carry

        lax.fori_loop(0, seg // _L, merge_body, 0)

        pltpu.sync_copy(mask_v.at[pl.ds(0, seg)],
                        out_hbm.at[pl.ds(c * n_pad + s * seg, seg)])

    return k(edge_index), n_pad


def _tc_matmul_body(x_ref, w_ref, b_ref, y_ref):
    y = jax.lax.dot_general(
        x_ref[...], w_ref[...], (((1,), (1,)), ((), ())),
        preferred_element_type=jnp.float32)
    y_ref[...] = jnp.maximum(y + b_ref[...][None, :], 0.0)


def _tc_mask_body(rows_per_core, y_ref, cnt_ref, o_ref):
    a = (cnt_ref[pl.ds(0, rows_per_core), :]
         + cnt_ref[pl.ds(rows_per_core, rows_per_core), :])
    at = jnp.transpose(a)
    full = _N // 128
    for r in range(full):
        col = at[:, r:r + 1] > 0.0
        o_ref[r * 128:(r + 1) * 128, :] = jnp.where(
            col, y_ref[r * 128:(r + 1) * 128, :], 0.0)
    rem = _N - full * 128
    if rem:
        col = at[:rem, full:full + 1] > 0.0
        o_ref[full * 128:, :] = jnp.where(col, y_ref[full * 128:, :], 0.0)


def kernel(x, p_t, edge_index, edge_weights, W_value, b_value):
    del p_t, edge_weights
    flat, n_pad = _sc_indegree_mask(edge_index)
    assert n_pad % 128 == 0
    rows_per_core = n_pad // 128
    cnt_rows = flat.reshape(flat.shape[0] // 128, 128)
    y = pl.pallas_call(
        _tc_matmul_body,
        out_shape=jax.ShapeDtypeStruct((_N, _D), jnp.float32),
    )(x, W_value, b_value)
    out = pl.pallas_call(
        functools.partial(_tc_mask_body, rows_per_core),
        out_shape=jax.ShapeDtypeStruct((_N, _D), jnp.float32),
    )(y, cnt_rows)
    return out


# speedup vs baseline: 1.0210x; 1.0210x over previous
import functools

import jax
import jax.numpy as jnp
from jax import lax
from jax.experimental import pallas as pl
from jax.experimental.pallas import tpu as pltpu
from jax.experimental.pallas import tpu_sc as plsc

_N = 10000
_E = 320000
_D = 128
_L = 16


def _sc_indegree_mask(edge_index):
    mesh = plsc.VectorSubcoreMesh(core_axis_name="c", subcore_axis_name="s")
    nc, ns = mesh.num_cores, mesh.num_subcores
    nw = nc * ns
    n_pad = ((_N + ns * _L - 1) // (ns * _L)) * (ns * _L)
    seg = n_pad // ns

    n_tiles = _E // 128
    tpw = -(-n_tiles // nw)
    lo_scale = n_tiles - tpw

    @functools.partial(
        pl.kernel,
        mesh=mesh,
        out_type=jax.ShapeDtypeStruct((nc * n_pad,), jnp.float32),
        compiler_params=pltpu.CompilerParams(needs_layout_passes=False),
        scratch_types=[
            pltpu.VMEM((2, tpw * 128), jnp.int32),
            pltpu.VMEM((n_pad,), jnp.float32),
            pltpu.VMEM((ns, seg), jnp.float32),
            pltpu.VMEM_SHARED((ns, ns, seg), jnp.float32),
            pltpu.SemaphoreType.DMA,
            pltpu.SemaphoreType.DMA,
            pltpu.SemaphoreType.DMA,
        ],
    )
    def k(ei_hbm, out_hbm, idx_v, mask_v, merge_v, stage_s,
          sem_a, sem_b, st_sem):
        c = lax.axis_index("c")
        s = lax.axis_index("s")
        wid = c * ns + s
        lo = wid * lo_scale // (nw - 1)
        zero16 = jnp.zeros((_L,), jnp.float32)
        one16 = jnp.ones((_L,), jnp.float32)

        ta = tpw // 2
        tb = tpw - ta
        cp_a = pltpu.async_copy(
            ei_hbm.at[:, pl.ds(lo * 128, ta * 128)],
            idx_v.at[:, pl.ds(0, ta * 128)], sem_a)
        cp_b = pltpu.async_copy(
            ei_hbm.at[:, pl.ds((lo + ta) * 128, tb * 128)],
            idx_v.at[:, pl.ds(ta * 128, tb * 128)], sem_b)

        def init_body(i, carry):
            for u in range(8):
                mask_v[pl.ds((i * 8 + u) * _L, _L)] = zero16
            return carry

        lax.fori_loop(0, n_pad // _L // 8, init_body, 0)

        def scat_body(i, carry):
            for u in range(8):
                idx = idx_v[1, pl.ds((i * 8 + u) * _L, _L)]
                plsc.store_scatter(mask_v, [idx], one16)
            return carry

        cp_a.wait()
        lax.fori_loop(0, ta * 128 // _L // 8, scat_body, 0)
        cp_b.wait()
        lax.fori_loop(ta * 128 // _L // 8, tpw * 128 // _L // 8,
                      scat_body, 0)

        copies = [
            pltpu.async_copy(mask_v.at[pl.ds(chunk * seg, seg)],
                             stage_s.at[chunk, s], st_sem)
            for chunk in range(ns)
        ]
        for cp in copies:
            cp.wait()
        plsc.subcore_barrier()
        pltpu.sync_copy(stage_s.at[s], merge_v)

        def merge_body(j, carry):
            acc = zero16
            for t in range(ns):
                acc = acc + merge_v[t, pl.ds(j * _L, _L)]
            mask_v[pl.ds(j * _L, _L)] = acc
            return carry

        lax.fori_loop(0, seg // _L, merge_body, 0)

        pltpu.sync_copy(mask_v.at[pl.ds(0, seg)],
                        out_hbm.at[pl.ds(c * n_pad + s * seg, seg)])

    return k(edge_index), n_pad


_NCHUNK = 5


def _tc_body(rows_per_core, x_hbm, w_ref, b_ref, cnt_ref, o_hbm,
             x_v, o_v, in_sems, out_sems):
    rows = _N // _NCHUNK
    in_cps = [
        pltpu.async_copy(x_hbm.at[pl.ds(k * rows, rows)],
                         x_v.at[pl.ds(k * rows, rows)], in_sems.at[k])
        for k in range(_NCHUNK)
    ]
    a = (cnt_ref[pl.ds(0, rows_per_core), :]
         + cnt_ref[pl.ds(rows_per_core, rows_per_core), :])
    at = jnp.transpose(a)
    for cp in in_cps:
        cp.wait()
    y = jax.lax.dot_general(
        x_v[...], w_ref[...], (((1,), (1,)), ((), ())),
        preferred_element_type=jnp.float32)
    y = jnp.maximum(y + b_ref[...][None, :], 0.0)
    full = _N // 128
    for r in range(full):
        col = at[:, r:r + 1] > 0.0
        o_v[r * 128:(r + 1) * 128, :] = jnp.where(
            col, y[r * 128:(r + 1) * 128, :], 0.0)
    rem = _N - full * 128
    if rem:
        col = at[:rem, full:full + 1] > 0.0
        o_v[full * 128:, :] = jnp.where(col, y[full * 128:, :], 0.0)
    out_cps = [
        pltpu.async_copy(o_v.at[pl.ds(k * rows, rows)],
                         o_hbm.at[pl.ds(k * rows, rows)], out_sems.at[k])
        for k in range(_NCHUNK)
    ]
    for cp in out_cps:
        cp.wait()


def kernel(x, p_t, edge_index, edge_weights, W_value, b_value):
    del p_t, edge_weights
    flat, n_pad = _sc_indegree_mask(edge_index)
    assert n_pad % 128 == 0
    rows_per_core = n_pad // 128
    cnt_rows = flat.reshape(flat.shape[0] // 128, 128)
    out = pl.pallas_call(
        functools.partial(_tc_body, rows_per_core),
        in_specs=[
            pl.BlockSpec(memory_space=pl.ANY),
            pl.BlockSpec(memory_space=pltpu.VMEM),
            pl.BlockSpec(memory_space=pltpu.VMEM),
            pl.BlockSpec(memory_space=pltpu.VMEM),
        ],
        out_specs=pl.BlockSpec(memory_space=pl.ANY),
        out_shape=jax.ShapeDtypeStruct((_N, _D), jnp.float32),
        scratch_shapes=[
            pltpu.VMEM((_N, _D), jnp.float32),
            pltpu.VMEM((_N, _D), jnp.float32),
            pltpu.SemaphoreType.DMA((_NCHUNK,)),
            pltpu.SemaphoreType.DMA((_NCHUNK,)),
        ],
    )(x, W_value, b_value, cnt_rows)
    return out


# speedup vs baseline: 1.0777x; 1.0556x over previous
import functools

import jax
import jax.numpy as jnp
from jax import lax
from jax.experimental import pallas as pl
from jax.experimental.pallas import tpu as pltpu
from jax.experimental.pallas import tpu_sc as plsc

_N = 10000
_E = 320000
_D = 128
_L = 16


def _sc_indegree_mask(edge_index):
    mesh = plsc.VectorSubcoreMesh(core_axis_name="c", subcore_axis_name="s")
    nc, ns = mesh.num_cores, mesh.num_subcores
    nw = nc * ns
    n_pad = ((_N + ns * _L - 1) // (ns * _L)) * (ns * _L)
    seg = n_pad // ns

    n_tiles = _E // 128
    tpw = -(-n_tiles // nw)
    lo_scale = n_tiles - tpw

    @functools.partial(
        pl.kernel,
        mesh=mesh,
        out_type=jax.ShapeDtypeStruct((nc * n_pad,), jnp.float32),
        compiler_params=pltpu.CompilerParams(needs_layout_passes=False),
        scratch_types=[
            pltpu.VMEM((2, tpw * 128), jnp.int32),
            pltpu.VMEM((n_pad,), jnp.float32),
            pltpu.VMEM((ns, seg), jnp.float32),
            pltpu.VMEM_SHARED((ns, ns, seg), jnp.float32),
            pltpu.SemaphoreType.DMA,
            pltpu.SemaphoreType.DMA,
            pltpu.SemaphoreType.DMA,
        ],
    )
    def k(ei_hbm, out_hbm, idx_v, mask_v, merge_v, stage_s,
          sem_a, sem_b, st_sem):
        c = lax.axis_index("c")
        s = lax.axis_index("s")
        wid = c * ns + s
        lo = wid * lo_scale // (nw - 1)
        zero16 = jnp.zeros((_L,), jnp.float32)
        one16 = jnp.ones((_L,), jnp.float32)

        ta = tpw // 2
        tb = tpw - ta
        cp_a = pltpu.async_copy(
            ei_hbm.at[:, pl.ds(lo * 128, ta * 128)],
            idx_v.at[:, pl.ds(0, ta * 128)], sem_a)
        cp_b = pltpu.async_copy(
            ei_hbm.at[:, pl.ds((lo + ta) * 128, tb * 128)],
            idx_v.at[:, pl.ds(ta * 128, tb * 128)], sem_b)

        def init_body(i, carry):
            for u in range(8):
                mask_v[pl.ds((i * 8 + u) * _L, _L)] = zero16
            return carry

        lax.fori_loop(0, n_pad // _L // 8, init_body, 0)

        def scat_body(i, carry):
            for u in range(8):
                idx = idx_v[1, pl.ds((i * 8 + u) * _L, _L)]
                plsc.store_scatter(mask_v, [idx], one16)
            return carry

        cp_a.wait()
        lax.fori_loop(0, ta * 128 // _L // 8, scat_body, 0)
        cp_b.wait()
        lax.fori_loop(ta * 128 // _L // 8, tpw * 128 // _L // 8,
                      scat_body, 0)

        copies = [
            pltpu.async_copy(mask_v.at[pl.ds(chunk * seg, seg)],
                             stage_s.at[chunk, s], st_sem)
            for chunk in range(ns)
        ]
        for cp in copies:
            cp.wait()
        plsc.subcore_barrier()
        pltpu.sync_copy(stage_s.at[s], merge_v)

        def merge_body(j, carry):
            acc = zero16
            for t in range(ns):
                acc = acc + merge_v[t, pl.ds(j * _L, _L)]
            mask_v[pl.ds(j * _L, _L)] = acc
            return carry

        lax.fori_loop(0, seg // _L, merge_body, 0)

        pltpu.sync_copy(mask_v.at[pl.ds(0, seg)],
                        out_hbm.at[pl.ds(c * n_pad + s * seg, seg)])

    return k(edge_index), n_pad


_NCHUNK = 5


_CHUNK = 2048


def _tc_body(rows_per_core, x_hbm, w_ref, b_ref, cnt_ref, o_hbm,
             x_v, o_v, in_sems, out_sems):
    nchunk = -(-_N // _CHUNK)
    in_cps = []
    for k in range(nchunk):
        lo = k * _CHUNK
        sz = min(_CHUNK, _N - lo)
        in_cps.append(pltpu.async_copy(
            x_hbm.at[pl.ds(lo, sz)], x_v.at[pl.ds(lo, sz)], in_sems.at[k]))
    a = (cnt_ref[pl.ds(0, rows_per_core), :]
         + cnt_ref[pl.ds(rows_per_core, rows_per_core), :])
    at = jnp.transpose(a)
    out_cps = []
    for k in range(nchunk):
        lo = k * _CHUNK
        sz = min(_CHUNK, _N - lo)
        in_cps[k].wait()
        y = jax.lax.dot_general(
            x_v[pl.ds(lo, sz)], w_ref[...], (((1,), (1,)), ((), ())),
            preferred_element_type=jnp.float32)
        y = jnp.maximum(y + b_ref[...][None, :], 0.0)
        for r in range(sz // 128):
            g = lo // 128 + r
            col = at[:, g:g + 1] > 0.0
            o_v[pl.ds(lo + r * 128, 128)] = jnp.where(
                col, y[r * 128:(r + 1) * 128, :], 0.0)
        rem = sz - (sz // 128) * 128
        if rem:
            g = lo // 128 + sz // 128
            col = at[:rem, g:g + 1] > 0.0
            o_v[pl.ds(lo + sz - rem, rem)] = jnp.where(
                col, y[sz - rem:, :], 0.0)
        out_cps.append(pltpu.async_copy(
            o_v.at[pl.ds(lo, sz)], o_hbm.at[pl.ds(lo, sz)], out_sems.at[k]))
    for cp in out_cps:
        cp.wait()


def kernel(x, p_t, edge_index, edge_weights, W_value, b_value):
    del p_t, edge_weights
    flat, n_pad = _sc_indegree_mask(edge_index)
    assert n_pad % 128 == 0
    rows_per_core = n_pad // 128
    cnt_rows = flat.reshape(flat.shape[0] // 128, 128)
    out = pl.pallas_call(
        functools.partial(_tc_body, rows_per_core),
        in_specs=[
            pl.BlockSpec(memory_space=pl.ANY),
            pl.BlockSpec(memory_space=pltpu.VMEM),
            pl.BlockSpec(memory_space=pltpu.VMEM),
            pl.BlockSpec(memory_space=pltpu.VMEM),
        ],
        out_specs=pl.BlockSpec(memory_space=pl.ANY),
        out_shape=jax.ShapeDtypeStruct((_N, _D), jnp.float32),
        scratch_shapes=[
            pltpu.VMEM((_N, _D), jnp.float32),
            pltpu.VMEM((_N, _D), jnp.float32),
            pltpu.SemaphoreType.DMA((_NCHUNK,)),
            pltpu.SemaphoreType.DMA((_NCHUNK,)),
        ],
    )(x, W_value, b_value, cnt_rows)
    return out


# speedup vs baseline: 1.2249x; 1.1366x over previous
import functools

import jax
import jax.numpy as jnp
from jax import lax
from jax.experimental import pallas as pl
from jax.experimental.pallas import tpu as pltpu
from jax.experimental.pallas import tpu_sc as plsc

_N = 10000
_E = 320000
_D = 128
_L = 16


def _sc_indegree_mask(edge_index):
    mesh = plsc.VectorSubcoreMesh(core_axis_name="c", subcore_axis_name="s")
    nc, ns = mesh.num_cores, mesh.num_subcores
    nw = nc * ns
    n_pad = ((_N + ns * _L - 1) // (ns * _L)) * (ns * _L)
    seg = n_pad // ns

    n_tiles = _E // 128
    tpw = -(-n_tiles // nw)
    lo_scale = n_tiles - tpw

    @functools.partial(
        pl.kernel,
        mesh=mesh,
        out_type=jax.ShapeDtypeStruct((nc * n_pad,), jnp.float32),
        compiler_params=pltpu.CompilerParams(needs_layout_passes=False),
        scratch_types=[
            pltpu.VMEM((2, tpw * 128), jnp.int32),
            pltpu.VMEM((n_pad,), jnp.float32),
            pltpu.VMEM((ns, 384), jnp.float32),
            pltpu.VMEM((ns, 256), jnp.float32),
            pltpu.VMEM_SHARED((ns, ns, 384), jnp.float32),
            pltpu.VMEM_SHARED((ns, ns, 256), jnp.float32),
            pltpu.SemaphoreType.DMA,
            pltpu.SemaphoreType.DMA,
            pltpu.SemaphoreType.DMA,
        ],
    )
    def k(ei_hbm, out_hbm, idx_v, mask_v, mv0, mv1, st0, st1,
          sem_a, sem_b, st_sem):
        c = lax.axis_index("c")
        s = lax.axis_index("s")
        wid = c * ns + s
        lo = wid * lo_scale // (nw - 1)
        zero16 = jnp.zeros((_L,), jnp.float32)
        one16 = jnp.ones((_L,), jnp.float32)

        ta = tpw // 2
        tb = tpw - ta
        cp_a = pltpu.async_copy(
            ei_hbm.at[:, pl.ds(lo * 128, ta * 128)],
            idx_v.at[:, pl.ds(0, ta * 128)], sem_a)
        cp_b = pltpu.async_copy(
            ei_hbm.at[:, pl.ds((lo + ta) * 128, tb * 128)],
            idx_v.at[:, pl.ds(ta * 128, tb * 128)], sem_b)

        @functools.partial(plsc.parallel_loop, 0, n_pad // _L, unroll=8)
        def _(i):
            mask_v[pl.ds(i * _L, _L)] = zero16

        def scat_body(v):
            idx = idx_v[1, pl.ds(v * _L, _L)]
            plsc.store_scatter(mask_v, [idx], one16)

        cp_a.wait()
        plsc.parallel_loop(0, ta * 8, unroll=8)(scat_body)
        cp_b.wait()
        plsc.parallel_loop(ta * 8, tpw * 8, unroll=8)(scat_body)

        copies = [
            pltpu.async_copy(mask_v.at[pl.ds(chunk * seg, 384)],
                             st0.at[chunk, s], st_sem)
            for chunk in range(ns)
        ] + [
            pltpu.async_copy(mask_v.at[pl.ds(chunk * seg + 384, 256)],
                             st1.at[chunk, s], st_sem)
            for chunk in range(ns)
        ]
        for cp in copies:
            cp.wait()
        plsc.subcore_barrier()

        cp0 = pltpu.async_copy(st0.at[s], mv0, sem_a)
        cp1 = pltpu.async_copy(st1.at[s], mv1, sem_b)

        cp0.wait()

        @functools.partial(plsc.parallel_loop, 0, 384 // _L, unroll=2)
        def _(j):
            acc = zero16
            for t in range(ns):
                acc = acc + mv0[t, pl.ds(j * _L, _L)]
            mask_v[pl.ds(j * _L, _L)] = acc

        cp1.wait()

        @functools.partial(plsc.parallel_loop, 0, 256 // _L, unroll=2)
        def _(j):
            acc = zero16
            for t in range(ns):
                acc = acc + mv1[t, pl.ds(j * _L, _L)]
            mask_v[pl.ds(384 + j * _L, _L)] = acc

        pltpu.sync_copy(mask_v.at[pl.ds(0, seg)],
                        out_hbm.at[pl.ds(c * n_pad + s * seg, seg)])

    return k(edge_index), n_pad


_CHUNK = 2048


def _tc_body(rows_per_core, x_hbm, w_ref, b_ref, cnt_ref, o_hbm,
             x_v, o_v, in_sems, out_sems):
    nchunk = -(-_N // _CHUNK)
    in_cps = []
    for k in range(nchunk):
        lo = k * _CHUNK
        sz = min(_CHUNK, _N - lo)
        in_cps.append(pltpu.async_copy(
            x_hbm.at[pl.ds(lo, sz)], x_v.at[pl.ds(lo, sz)], in_sems.at[k]))
    a = (cnt_ref[pl.ds(0, rows_per_core), :]
         + cnt_ref[pl.ds(rows_per_core, rows_per_core), :])
    at = jnp.transpose(a)
    out_cps = []
    for k in range(nchunk):
        lo = k * _CHUNK
        sz = min(_CHUNK, _N - lo)
        in_cps[k].wait()
        y = jax.lax.dot_general(
            x_v[pl.ds(lo, sz)], w_ref[...], (((1,), (1,)), ((), ())),
            preferred_element_type=jnp.float32)
        y = jnp.maximum(y + b_ref[...][None, :], 0.0)
        for r in range(sz // 128):
            g = lo // 128 + r
            col = at[:, g:g + 1] > 0.0
            o_v[pl.ds(lo + r * 128, 128)] = jnp.where(
                col, y[r * 128:(r + 1) * 128, :], 0.0)
        rem = sz - (sz // 128) * 128
        if rem:
            g = lo // 128 + sz // 128
            col = at[:rem, g:g + 1] > 0.0
            o_v[pl.ds(lo + sz - rem, rem)] = jnp.where(
                col, y[sz - rem:, :], 0.0)
        out_cps.append(pltpu.async_copy(
            o_v.at[pl.ds(lo, sz)], o_hbm.at[pl.ds(lo, sz)], out_sems.at[k]))
    for cp in out_cps:
        cp.wait()


def kernel(x, p_t, edge_index, edge_weights, W_value, b_value):
    del p_t, edge_weights
    flat, n_pad = _sc_indegree_mask(edge_index)
    assert n_pad % 128 == 0
    rows_per_core = n_pad // 128
    cnt_rows = flat.reshape(flat.shape[0] // 128, 128)
    out = pl.pallas_call(
        functools.partial(_tc_body, rows_per_core),
        in_specs=[
            pl.BlockSpec(memory_space=pl.ANY),
            pl.BlockSpec(memory_space=pltpu.VMEM),
            pl.BlockSpec(memory_space=pltpu.VMEM),
            pl.BlockSpec(memory_space=pltpu.VMEM),
        ],
        out_specs=pl.BlockSpec(memory_space=pl.ANY),
        out_shape=jax.ShapeDtypeStruct((_N, _D), jnp.float32),
        scratch_shapes=[
            pltpu.VMEM((_N, _D), jnp.float32),
            pltpu.VMEM((_N, _D), jnp.float32),
            pltpu.SemaphoreType.DMA((-(-_N // _CHUNK),)),
            pltpu.SemaphoreType.DMA((-(-_N // _CHUNK),)),
        ],
    )(x, W_value, b_value, cnt_rows)
    return out
